# Initial kernel scaffold; baseline (speedup 1.0000x reference)
import jax
import jax.numpy as jnp
from jax.experimental import pallas as pl


def kernel(node_features, edge_index, batch, W0, b0, g0, be0, W1, b1, g1, be1, W2, b2, g2, be2, Wc, bc):
    raise NotImplementedError("write your pallas kernel here")


# speedup vs baseline: 6.8751x; 6.8751x over previous
import functools

import jax
import jax.numpy as jnp
from jax import lax
from jax.experimental import pallas as pl
from jax.experimental.pallas import tpu as pltpu
from jax.experimental.pallas import tpu_sc as plsc

N = 10000
E = 320000
D = 128
H = 128
C = 3
G = 64
EPS = 1e-5

NC = 2
NS = 16
NW = NC * NS
NP = 10240
RPT = NP // NS

E_PAD = 327680
ER = E_PAD // 128
RW = ER // NW
IB = 8

_mesh = plsc.VectorSubcoreMesh(
    core_axis_name="c", subcore_axis_name="s", num_cores=NC, num_subcores=NS)


def _zero_fill(zbuf, rows):
    for i in range(rows):
        for j in range(H // 16):
            zbuf[i, pl.ds(j * 16, 16)] = jnp.zeros((16,), jnp.float32)


@functools.partial(
    pl.kernel,
    out_type=jax.ShapeDtypeStruct((NC, 128, 128), jnp.float32),
    mesh=_mesh,
    compiler_params=pltpu.CompilerParams(needs_layout_passes=False),
    scratch_types=[
        pltpu.VMEM((IB, 128), jnp.int32),
        pltpu.VMEM((128, 128), jnp.float32),
        pltpu.VMEM((16, 128), jnp.float32),
        pltpu.VMEM((1, 128), jnp.int32),
        pltpu.VMEM_SHARED((128, 128), jnp.float32),
    ],
)
def _deg_call(dst_hbm, out_hbm, didx, deg2, zbuf, ident, acc):
    c = lax.axis_index("c")
    s = lax.axis_index("s")
    w = c * NS + s
    for i in range(16):
        for k in range(8):
            zbuf[i, pl.ds(k * 16, 16)] = jnp.zeros((16,), jnp.float32)
    for k in range(8):
        ident[0, pl.ds(k * 16, 16)] = lax.iota(jnp.int32, 16) + 16 * k

    def zloop(i, carry):
        for k in range(8):
            deg2[i, pl.ds(k * 16, 16)] = jnp.zeros((16,), jnp.float32)
        return carry

    lax.fori_loop(0, 128, zloop, 0)

    @pl.when(s < 8)
    def _():
        pltpu.sync_copy(zbuf, acc.at[pl.ds(16 * s, 16)])
    plsc.subcore_barrier()

    ones16 = jnp.ones((16,), jnp.float32)

    def body(gi, carry):
        pltpu.sync_copy(dst_hbm.at[pl.ds(w * RW + gi * IB, IB)], didx)
        for j in range(IB):
            for k in range(8):
                n = didx[j, pl.ds(k * 16, 16)]
                plsc.addupdate_scatter(
                    deg2, [lax.shift_right_logical(n, 7), n & 127], ones16)
        return carry

    lax.fori_loop(0, RW // IB, body, 0)

    pltpu.sync_copy(deg2, acc.at[ident.at[0]], add=True)
    plsc.subcore_barrier()

    @pl.when(s < 8)
    def _():
        pltpu.sync_copy(acc.at[pl.ds(16 * s, 16)],
                        out_hbm.at[c, pl.ds(16 * s, 16)])


def _agg_body(y_hbm, src_hbm, dst_hbm, out_hbm, sidx, didx, rows, zbuf, acc,
              sem):
    c = lax.axis_index("c")
    s = lax.axis_index("s")
    w = c * NS + s
    _zero_fill(zbuf, 16)

    def zloop(i, carry):
        pltpu.sync_copy(zbuf, acc.at[pl.ds(s * RPT + i * 16, 16)])
        return carry

    lax.fori_loop(0, RPT // 16, zloop, 0)
    plsc.subcore_barrier()

    def body(gi, carry):
        r0 = w * RW + gi * IB
        pltpu.sync_copy(src_hbm.at[pl.ds(r0, IB)], sidx)
        pltpu.sync_copy(dst_hbm.at[pl.ds(r0, IB)], didx)
        for j in range(IB):
            pltpu.async_copy(y_hbm.at[sidx.at[j]], rows, sem).wait()
            pltpu.sync_copy(rows, acc.at[didx.at[j]], add=True)
        return carry

    lax.fori_loop(0, RW // IB, body, 0)
    plsc.subcore_barrier()
    pltpu.sync_copy(acc.at[pl.ds(s * RPT, RPT)],
                    out_hbm.at[c, pl.ds(s * RPT, RPT)])


_agg_call = functools.partial(
    pl.kernel,
    out_type=jax.ShapeDtypeStruct((NC, NP, H), jnp.float32),
    mesh=_mesh,
    compiler_params=pltpu.CompilerParams(needs_layout_passes=False),
    scratch_types=[
        pltpu.VMEM((IB, 128), jnp.int32),
        pltpu.VMEM((IB, 128), jnp.int32),
        pltpu.VMEM((128, H), jnp.float32),
        pltpu.VMEM((16, H), jnp.float32),
        pltpu.VMEM_SHARED((NP, H), jnp.float32),
        pltpu.SemaphoreType.DMA,
    ],
)(_agg_body)


def _tc0_body(dg0_ref, dg1_ref, x_ref, w_ref, dinv_ref, y_ref):
    deg = dg0_ref[...] + dg1_ref[...] + 1.0
    dinv = lax.rsqrt(deg)
    h = jnp.dot(x_ref[...], w_ref[...], preferred_element_type=jnp.float32)
    dinv_ref[...] = dinv
    y_ref[...] = h * dinv


def _bn_relu(a_ref, y_ref, dinv_ref, b_ref, g_ref, be_ref):
    a = a_ref[0, :N, :] + a_ref[1, :N, :]
    dinv = dinv_ref[...]
    t = dinv * (a + y_ref[...]) + b_ref[...]
    mu = jnp.mean(t, axis=0, keepdims=True)
    xc = t - mu
    var = jnp.mean(xc * xc, axis=0, keepdims=True)
    return jnp.maximum(xc * lax.rsqrt(var + EPS) * g_ref[...] + be_ref[...],
                       0.0)


def _tc_mid_body(a_ref, y_ref, dinv_ref, b_ref, g_ref, be_ref, w_ref, yn_ref):
    h = _bn_relu(a_ref, y_ref, dinv_ref, b_ref, g_ref, be_ref)
    yn_ref[...] = jnp.dot(
        h, w_ref[...], preferred_element_type=jnp.float32) * dinv_ref[...]


def _tc_fin_body(a_ref, y_ref, dinv_ref, b_ref, g_ref, be_ref, batch_ref,
                 wc_ref, bc_ref, o_ref):
    h = _bn_relu(a_ref, y_ref, dinv_ref, b_ref, g_ref, be_ref)
    onehot = (lax.broadcasted_iota(jnp.int32, (G, N), 0)
              == batch_ref[...]).astype(jnp.float32)
    sums = jnp.dot(onehot, h, preferred_element_type=jnp.float32)
    cnts = jnp.sum(onehot, axis=1, keepdims=True)
    pooled = sums / jnp.maximum(cnts, 1.0)
    o_ref[...] = jnp.dot(
        pooled, wc_ref[...], preferred_element_type=jnp.float32) + bc_ref[...]


_tc0 = pl.pallas_call(
    _tc0_body,
    out_shape=(jax.ShapeDtypeStruct((N, 1), jnp.float32),
               jax.ShapeDtypeStruct((N, H), jnp.float32)))

_tc_mid = pl.pallas_call(
    _tc_mid_body, out_shape=jax.ShapeDtypeStruct((N, H), jnp.float32))

_tc_fin = pl.pallas_call(
    _tc_fin_body, out_shape=jax.ShapeDtypeStruct((G, 128), jnp.float32))


def kernel(node_features, edge_index, batch, W0, b0, g0, be0, W1, b1, g1, be1,
           W2, b2, g2, be2, Wc, bc):
    src = edge_index[0].astype(jnp.int32)
    dst = edge_index[1].astype(jnp.int32)
    pad = E_PAD - E
    src_p = jnp.concatenate([src, jnp.zeros((pad,), jnp.int32)]).reshape(
        ER, 128)
    dst_p = jnp.concatenate([dst, jnp.full((pad,), N, jnp.int32)]).reshape(
        ER, 128)
    batch2d = batch.astype(jnp.int32).reshape(1, N)
    wcp = jnp.pad(Wc, ((0, 0), (0, 128 - C)))
    bcp = jnp.pad(bc, (0, 128 - C)).reshape(1, 128)

    deg_raw = _deg_call(dst_p).reshape(NC, 128 * 128, 1)
    dinv, y = _tc0(deg_raw[0, :N], deg_raw[1, :N], node_features, W0)
    for b, g, be, Wn in ((b0, g0, be0, W1), (b1, g1, be1, W2)):
        agg = _agg_call(y, src_p, dst_p)
        y = _tc_mid(agg, y, dinv, b.reshape(1, H), g.reshape(1, H),
                    be.reshape(1, H), Wn)
    agg = _agg_call(y, src_p, dst_p)
    out = _tc_fin(agg, y, dinv, b2.reshape(1, H), g2.reshape(1, H),
                  be2.reshape(1, H), batch2d, wcp, bcp)
    return out[:, :C]


# speedup vs baseline: 7.5038x; 1.0914x over previous
import functools

import jax
import jax.numpy as jnp
from jax import lax
from jax.experimental import pallas as pl
from jax.experimental.pallas import tpu as pltpu
from jax.experimental.pallas import tpu_sc as plsc

N = 10000
E = 320000
D = 128
H = 128
C = 3
G = 64
EPS = 1e-5

NC = 2
NS = 16
NW = NC * NS
NP = 10240
RPT = NP // NS

E_PAD = 327680
ER = E_PAD // 128
RW = ER // NW
IB = 8

_mesh = plsc.VectorSubcoreMesh(
    core_axis_name="c", subcore_axis_name="s", num_cores=NC, num_subcores=NS)


def _zero_fill(zbuf, rows):
    for i in range(rows):
        for j in range(H // 16):
            zbuf[i, pl.ds(j * 16, 16)] = jnp.zeros((16,), jnp.float32)


@functools.partial(
    pl.kernel,
    out_type=jax.ShapeDtypeStruct((NC, 128, 128), jnp.float32),
    mesh=_mesh,
    compiler_params=pltpu.CompilerParams(needs_layout_passes=False),
    scratch_types=[
        pltpu.VMEM((IB, 128), jnp.int32),
        pltpu.VMEM((128, 128), jnp.float32),
        pltpu.VMEM((16, 128), jnp.float32),
        pltpu.VMEM((1, 128), jnp.int32),
        pltpu.VMEM_SHARED((128, 128), jnp.float32),
    ],
)
def _deg_call(dst_hbm, out_hbm, didx, deg2, zbuf, ident, acc):
    c = lax.axis_index("c")
    s = lax.axis_index("s")
    w = c * NS + s
    for i in range(16):
        for k in range(8):
            zbuf[i, pl.ds(k * 16, 16)] = jnp.zeros((16,), jnp.float32)
    for k in range(8):
        ident[0, pl.ds(k * 16, 16)] = lax.iota(jnp.int32, 16) + 16 * k

    def zloop(i, carry):
        for k in range(8):
            deg2[i, pl.ds(k * 16, 16)] = jnp.zeros((16,), jnp.float32)
        return carry

    lax.fori_loop(0, 128, zloop, 0)

    @pl.when(s < 8)
    def _():
        pltpu.sync_copy(zbuf, acc.at[pl.ds(16 * s, 16)])
    plsc.subcore_barrier()

    ones16 = jnp.ones((16,), jnp.float32)

    def body(gi, carry):
        pltpu.sync_copy(dst_hbm.at[pl.ds(w * RW + gi * IB, IB)], didx)
        for j in range(IB):
            for k in range(8):
                n = didx[j, pl.ds(k * 16, 16)]
                plsc.addupdate_scatter(
                    deg2, [lax.shift_right_logical(n, 7), n & 127], ones16)
        return carry

    lax.fori_loop(0, RW // IB, body, 0)

    pltpu.sync_copy(deg2, acc.at[ident.at[0]], add=True)
    plsc.subcore_barrier()

    @pl.when(s < 8)
    def _():
        pltpu.sync_copy(acc.at[pl.ds(16 * s, 16)],
                        out_hbm.at[c, pl.ds(16 * s, 16)])


def _agg_body(y_hbm, src_hbm, dst_hbm, out_hbm, sidx, didx, rows0, rows1,
              zbuf, acc, sem0, sem1):
    c = lax.axis_index("c")
    s = lax.axis_index("s")
    w = c * NS + s
    _zero_fill(zbuf, 16)

    def zloop(i, carry):
        pltpu.sync_copy(zbuf, acc.at[pl.ds(s * RPT + i * 16, 16)])
        return carry

    lax.fori_loop(0, RPT // 16, zloop, 0)
    plsc.subcore_barrier()

    bufs = (rows0, rows1)
    sems = (sem0, sem1)

    def body(gi, carry):
        r0 = w * RW + gi * IB
        pltpu.sync_copy(src_hbm.at[pl.ds(r0, IB)], sidx)
        pltpu.sync_copy(dst_hbm.at[pl.ds(r0, IB)], didx)
        pending = pltpu.async_copy(y_hbm.at[sidx.at[0]], bufs[0], sems[0])
        for j in range(IB):
            if j + 1 < IB:
                nxt = pltpu.async_copy(
                    y_hbm.at[sidx.at[j + 1]], bufs[(j + 1) % 2],
                    sems[(j + 1) % 2])
            pending.wait()
            pltpu.sync_copy(bufs[j % 2], acc.at[didx.at[j]], add=True)
            if j + 1 < IB:
                pending = nxt
        return carry

    lax.fori_loop(0, RW // IB, body, 0)
    plsc.subcore_barrier()
    pltpu.sync_copy(acc.at[pl.ds(s * RPT, RPT)],
                    out_hbm.at[c, pl.ds(s * RPT, RPT)])


_agg_call = functools.partial(
    pl.kernel,
    out_type=jax.ShapeDtypeStruct((NC, NP, H), jnp.float32),
    mesh=_mesh,
    compiler_params=pltpu.CompilerParams(needs_layout_passes=False),
    scratch_types=[
        pltpu.VMEM((IB, 128), jnp.int32),
        pltpu.VMEM((IB, 128), jnp.int32),
        pltpu.VMEM((128, H), jnp.float32),
        pltpu.VMEM((128, H), jnp.float32),
        pltpu.VMEM((16, H), jnp.float32),
        pltpu.VMEM_SHARED((NP, H), jnp.float32),
        pltpu.SemaphoreType.DMA,
        pltpu.SemaphoreType.DMA,
    ],
)(_agg_body)


def _tc0_body(dg0_ref, dg1_ref, x_ref, w_ref, dinv_ref, y_ref):
    deg = dg0_ref[...] + dg1_ref[...] + 1.0
    dinv = lax.rsqrt(deg)
    h = jnp.dot(x_ref[...], w_ref[...], preferred_element_type=jnp.float32)
    dinv_ref[...] = dinv
    y_ref[...] = h * dinv


def _bn_relu(a_ref, y_ref, dinv_ref, b_ref, g_ref, be_ref):
    a = a_ref[0, :N, :] + a_ref[1, :N, :]
    dinv = dinv_ref[...]
    t = dinv * (a + y_ref[...]) + b_ref[...]
    mu = jnp.mean(t, axis=0, keepdims=True)
    xc = t - mu
    var = jnp.mean(xc * xc, axis=0, keepdims=True)
    return jnp.maximum(xc * lax.rsqrt(var + EPS) * g_ref[...] + be_ref[...],
                       0.0)


def _tc_mid_body(a_ref, y_ref, dinv_ref, b_ref, g_ref, be_ref, w_ref, yn_ref):
    h = _bn_relu(a_ref, y_ref, dinv_ref, b_ref, g_ref, be_ref)
    yn_ref[...] = jnp.dot(
        h, w_ref[...], preferred_element_type=jnp.float32) * dinv_ref[...]


def _tc_fin_body(a_ref, y_ref, dinv_ref, b_ref, g_ref, be_ref, batch_ref,
                 wc_ref, bc_ref, o_ref):
    h = _bn_relu(a_ref, y_ref, dinv_ref, b_ref, g_ref, be_ref)
    onehot = (lax.broadcasted_iota(jnp.int32, (G, N), 0)
              == batch_ref[...]).astype(jnp.float32)
    sums = jnp.dot(onehot, h, preferred_element_type=jnp.float32)
    cnts = jnp.sum(onehot, axis=1, keepdims=True)
    pooled = sums / jnp.maximum(cnts, 1.0)
    o_ref[...] = jnp.dot(
        pooled, wc_ref[...], preferred_element_type=jnp.float32) + bc_ref[...]


_tc0 = pl.pallas_call(
    _tc0_body,
    out_shape=(jax.ShapeDtypeStruct((N, 1), jnp.float32),
               jax.ShapeDtypeStruct((N, H), jnp.float32)))

_tc_mid = pl.pallas_call(
    _tc_mid_body, out_shape=jax.ShapeDtypeStruct((N, H), jnp.float32))

_tc_fin = pl.pallas_call(
    _tc_fin_body, out_shape=jax.ShapeDtypeStruct((G, 128), jnp.float32))


def kernel(node_features, edge_index, batch, W0, b0, g0, be0, W1, b1, g1, be1,
           W2, b2, g2, be2, Wc, bc):
    src = edge_index[0].astype(jnp.int32)
    dst = edge_index[1].astype(jnp.int32)
    pad = E_PAD - E
    src_p = jnp.concatenate([src, jnp.zeros((pad,), jnp.int32)]).reshape(
        ER, 128)
    dst_p = jnp.concatenate([dst, jnp.full((pad,), N, jnp.int32)]).reshape(
        ER, 128)
    batch2d = batch.astype(jnp.int32).reshape(1, N)
    wcp = jnp.pad(Wc, ((0, 0), (0, 128 - C)))
    bcp = jnp.pad(bc, (0, 128 - C)).reshape(1, 128)

    deg_raw = _deg_call(dst_p).reshape(NC, 128 * 128, 1)
    dinv, y = _tc0(deg_raw[0, :N], deg_raw[1, :N], node_features, W0)
    for b, g, be, Wn in ((b0, g0, be0, W1), (b1, g1, be1, W2)):
        agg = _agg_call(y, src_p, dst_p)
        y = _tc_mid(agg, y, dinv, b.reshape(1, H), g.reshape(1, H),
                    be.reshape(1, H), Wn)
    agg = _agg_call(y, src_p, dst_p)
    out = _tc_fin(agg, y, dinv, b2.reshape(1, H), g2.reshape(1, H),
                  be2.reshape(1, H), batch2d, wcp, bcp)
    return out[:, :C]


# speedup vs baseline: 8.2899x; 1.1048x over previous
import functools

import jax
import jax.numpy as jnp
from jax import lax
from jax.experimental import pallas as pl
from jax.experimental.pallas import tpu as pltpu
from jax.experimental.pallas import tpu_sc as plsc

N = 10000
E = 320000
D = 128
H = 128
C = 3
G = 64
EPS = 1e-5

NC = 2
NS = 16
NW = NC * NS
NP = 10240
RPT = NP // NS

E_PAD = 327680
ER = E_PAD // 128
RW = ER // NW
IB = 8

FAST_CORE = 0
FAST_ROWS = 2048
SLOW_ROWS = ER - FAST_ROWS
FAST_PT = FAST_ROWS // NS
SLOW_PT = SLOW_ROWS // NS

_mesh = plsc.VectorSubcoreMesh(
    core_axis_name="c", subcore_axis_name="s", num_cores=NC, num_subcores=NS)


def _zero_fill(zbuf, rows):
    for i in range(rows):
        for j in range(H // 16):
            zbuf[i, pl.ds(j * 16, 16)] = jnp.zeros((16,), jnp.float32)


@functools.partial(
    pl.kernel,
    out_type=jax.ShapeDtypeStruct((NC, 128, 128), jnp.float32),
    mesh=_mesh,
    compiler_params=pltpu.CompilerParams(needs_layout_passes=False),
    scratch_types=[
        pltpu.VMEM((IB, 128), jnp.int32),
        pltpu.VMEM((128, 128), jnp.float32),
        pltpu.VMEM((16, 128), jnp.float32),
        pltpu.VMEM((1, 128), jnp.int32),
        pltpu.VMEM_SHARED((128, 128), jnp.float32),
    ],
)
def _deg_call(dst_hbm, out_hbm, didx, deg2, zbuf, ident, acc):
    c = lax.axis_index("c")
    s = lax.axis_index("s")
    w = c * NS + s
    for i in range(16):
        for k in range(8):
            zbuf[i, pl.ds(k * 16, 16)] = jnp.zeros((16,), jnp.float32)
    for k in range(8):
        ident[0, pl.ds(k * 16, 16)] = lax.iota(jnp.int32, 16) + 16 * k

    def zloop(i, carry):
        for k in range(8):
            deg2[i, pl.ds(k * 16, 16)] = jnp.zeros((16,), jnp.float32)
        return carry

    lax.fori_loop(0, 128, zloop, 0)

    @pl.when(s < 8)
    def _():
        pltpu.sync_copy(zbuf, acc.at[pl.ds(16 * s, 16)])
    plsc.subcore_barrier()

    ones16 = jnp.ones((16,), jnp.float32)

    def body(gi, carry):
        pltpu.sync_copy(dst_hbm.at[pl.ds(w * RW + gi * IB, IB)], didx)
        for j in range(IB):
            for k in range(8):
                n = didx[j, pl.ds(k * 16, 16)]
                plsc.addupdate_scatter(
                    deg2, [lax.shift_right_logical(n, 7), n & 127], ones16)
        return carry

    lax.fori_loop(0, RW // IB, body, 0)

    pltpu.sync_copy(deg2, acc.at[ident.at[0]], add=True)
    plsc.subcore_barrier()

    @pl.when(s < 8)
    def _():
        pltpu.sync_copy(acc.at[pl.ds(16 * s, 16)],
                        out_hbm.at[c, pl.ds(16 * s, 16)])


def _agg_body(y_hbm, src_hbm, dst_hbm, out_hbm, sidx, didx, rows0, rows1,
              zbuf, acc, sem0, sem1):
    c = lax.axis_index("c")
    s = lax.axis_index("s")
    w = c * NS + s
    _zero_fill(zbuf, 16)

    def zloop(i, carry):
        pltpu.sync_copy(zbuf, acc.at[pl.ds(s * RPT + i * 16, 16)])
        return carry

    lax.fori_loop(0, RPT // 16, zloop, 0)
    plsc.subcore_barrier()

    bufs = (rows0, rows1)
    sems = (sem0, sem1)

    is_fast = c == FAST_CORE
    base = jnp.where(is_fast, s * FAST_PT,
                     FAST_ROWS + s * SLOW_PT).astype(jnp.int32)
    n_iter = jnp.where(is_fast, FAST_PT // IB, SLOW_PT // IB)

    def body(gi, carry):
        r0 = base + gi * IB
        pltpu.sync_copy(src_hbm.at[pl.ds(r0, IB)], sidx)
        pltpu.sync_copy(dst_hbm.at[pl.ds(r0, IB)], didx)
        pending = pltpu.async_copy(y_hbm.at[sidx.at[0]], bufs[0], sems[0])
        for j in range(IB):
            if j + 1 < IB:
                nxt = pltpu.async_copy(
                    y_hbm.at[sidx.at[j + 1]], bufs[(j + 1) % 2],
                    sems[(j + 1) % 2])
            pending.wait()
            pltpu.sync_copy(bufs[j % 2], acc.at[didx.at[j]], add=True)
            if j + 1 < IB:
                pending = nxt
        return carry

    lax.fori_loop(0, n_iter, body, 0)
    plsc.subcore_barrier()
    pltpu.sync_copy(acc.at[pl.ds(s * RPT, RPT)],
                    out_hbm.at[c, pl.ds(s * RPT, RPT)])


_agg_call = functools.partial(
    pl.kernel,
    out_type=jax.ShapeDtypeStruct((NC, NP, H), jnp.float32),
    mesh=_mesh,
    compiler_params=pltpu.CompilerParams(needs_layout_passes=False),
    scratch_types=[
        pltpu.VMEM((IB, 128), jnp.int32),
        pltpu.VMEM((IB, 128), jnp.int32),
        pltpu.VMEM((128, H), jnp.float32),
        pltpu.VMEM((128, H), jnp.float32),
        pltpu.VMEM((16, H), jnp.float32),
        pltpu.VMEM_SHARED((NP, H), jnp.float32),
        pltpu.SemaphoreType.DMA,
        pltpu.SemaphoreType.DMA,
    ],
)(_agg_body)


def _tc0_body(dg0_ref, dg1_ref, x_ref, w_ref, dinv_ref, y_ref):
    deg = dg0_ref[...] + dg1_ref[...] + 1.0
    dinv = lax.rsqrt(deg)
    h = jnp.dot(x_ref[...], w_ref[...], preferred_element_type=jnp.float32)
    dinv_ref[...] = dinv
    y_ref[...] = h * dinv


def _bn_relu(a_ref, y_ref, dinv_ref, b_ref, g_ref, be_ref):
    a = a_ref[0, :N, :] + a_ref[1, :N, :]
    dinv = dinv_ref[...]
    t = dinv * (a + y_ref[...]) + b_ref[...]
    mu = jnp.mean(t, axis=0, keepdims=True)
    xc = t - mu
    var = jnp.mean(xc * xc, axis=0, keepdims=True)
    return jnp.maximum(xc * lax.rsqrt(var + EPS) * g_ref[...] + be_ref[...],
                       0.0)


def _tc_mid_body(a_ref, y_ref, dinv_ref, b_ref, g_ref, be_ref, w_ref, yn_ref):
    h = _bn_relu(a_ref, y_ref, dinv_ref, b_ref, g_ref, be_ref)
    yn_ref[...] = jnp.dot(
        h, w_ref[...], preferred_element_type=jnp.float32) * dinv_ref[...]


def _tc_fin_body(a_ref, y_ref, dinv_ref, b_ref, g_ref, be_ref, batch_ref,
                 wc_ref, bc_ref, o_ref):
    h = _bn_relu(a_ref, y_ref, dinv_ref, b_ref, g_ref, be_ref)
    onehot = (lax.broadcasted_iota(jnp.int32, (G, N), 0)
              == batch_ref[...]).astype(jnp.float32)
    sums = jnp.dot(onehot, h, preferred_element_type=jnp.float32)
    cnts = jnp.sum(onehot, axis=1, keepdims=True)
    pooled = sums / jnp.maximum(cnts, 1.0)
    o_ref[...] = jnp.dot(
        pooled, wc_ref[...], preferred_element_type=jnp.float32) + bc_ref[...]


_tc0 = pl.pallas_call(
    _tc0_body,
    out_shape=(jax.ShapeDtypeStruct((N, 1), jnp.float32),
               jax.ShapeDtypeStruct((N, H), jnp.float32)))

_tc_mid = pl.pallas_call(
    _tc_mid_body, out_shape=jax.ShapeDtypeStruct((N, H), jnp.float32))

_tc_fin = pl.pallas_call(
    _tc_fin_body, out_shape=jax.ShapeDtypeStruct((G, 128), jnp.float32))


def kernel(node_features, edge_index, batch, W0, b0, g0, be0, W1, b1, g1, be1,
           W2, b2, g2, be2, Wc, bc):
    src = edge_index[0].astype(jnp.int32)
    dst = edge_index[1].astype(jnp.int32)
    pad = E_PAD - E
    src_p = jnp.concatenate([src, jnp.zeros((pad,), jnp.int32)]).reshape(
        ER, 128)
    dst_p = jnp.concatenate([dst, jnp.full((pad,), N, jnp.int32)]).reshape(
        ER, 128)
    batch2d = batch.astype(jnp.int32).reshape(1, N)
    wcp = jnp.pad(Wc, ((0, 0), (0, 128 - C)))
    bcp = jnp.pad(bc, (0, 128 - C)).reshape(1, 128)

    deg_raw = _deg_call(dst_p).reshape(NC, 128 * 128, 1)
    dinv, y = _tc0(deg_raw[0, :N], deg_raw[1, :N], node_features, W0)
    for b, g, be, Wn in ((b0, g0, be0, W1), (b1, g1, be1, W2)):
        agg = _agg_call(y, src_p, dst_p)
        y = _tc_mid(agg, y, dinv, b.reshape(1, H), g.reshape(1, H),
                    be.reshape(1, H), Wn)
    agg = _agg_call(y, src_p, dst_p)
    out = _tc_fin(agg, y, dinv, b2.reshape(1, H), g2.reshape(1, H),
                  be2.reshape(1, H), batch2d, wcp, bcp)
    return out[:, :C]
